# Initial kernel scaffold; baseline (speedup 1.0000x reference)
import jax
import jax.numpy as jnp
from jax.experimental import pallas as pl


def kernel(heatmap):
    raise NotImplementedError("write your pallas kernel here")


# speedup vs baseline: 2.9198x; 2.9198x over previous
import functools

import jax
import jax.numpy as jnp
from jax import lax
from jax.experimental import pallas as pl
from jax.experimental.pallas import tpu as pltpu

_K = 512
_H = 512
_W = 512
_HC = 64
_BORDER = 16
_HI = jax.lax.Precision.HIGHEST


def _shift0(x, d, fill):
    if d == 0:
        return x
    f = jnp.full((abs(d),) + x.shape[1:], fill, x.dtype)
    if d > 0:
        return jnp.concatenate([x[d:], f], axis=0)
    return jnp.concatenate([f, x[:d]], axis=0)


def _shift1(x, d, fill):
    if d == 0:
        return x
    f = jnp.full((x.shape[0], abs(d)) + x.shape[2:], fill, x.dtype)
    if d > 0:
        return jnp.concatenate([x[:, d:], f], axis=1)
    return jnp.concatenate([f, x[:, :d]], axis=1)


def _win17(x, shift):
    ninf = jnp.float32(-jnp.inf)
    w = x
    for d in range(1, 9):
        w = jnp.maximum(w, shift(x, d, ninf))
        w = jnp.maximum(w, shift(x, -d, ninf))
    return w


def _sum65(es):
    acc = es[0]
    for x in es[1:]:
        acc = acc + x
    return acc


def _kernel_body(h_ref, x_ref, y_ref, s_ref):
    h = h_ref[0]

    m = jnp.max(h, axis=0)
    e = jnp.exp(h - m[None])
    s = _sum65([e[c] for c in range(65)])
    prob = e[:64] / s[None]

    r512 = lax.broadcasted_iota(jnp.int32, (_H, _HC), 0)
    c64 = lax.broadcasted_iota(jnp.int32, (_H, _HC), 1)
    r64 = lax.broadcasted_iota(jnp.int32, (_HC, _W), 0)
    c512 = lax.broadcasted_iota(jnp.int32, (_HC, _W), 1)
    score = jnp.zeros((_H, _W), jnp.float32)
    for i in range(8):
        mi = jnp.zeros((_HC, _W), jnp.float32)
        for j in range(8):
            cjt = (c512 == 8 * r64 + j).astype(jnp.float32)
            mi = mi + jax.lax.dot(prob[8 * i + j], cjt, precision=_HI)
        ri = (r512 == 8 * c64 + i).astype(jnp.float32)
        score = score + jax.lax.dot(ri, mi, precision=_HI)

    ri2 = lax.broadcasted_iota(jnp.int32, (_H, _W), 0)
    ci2 = lax.broadcasted_iota(jnp.int32, (_H, _W), 1)
    inb = ((ri2 >= _BORDER) & (ri2 < _H - _BORDER)
           & (ci2 >= _BORDER) & (ci2 < _W - _BORDER))
    score = jnp.where(inb, score, 0.0)
    z = jnp.float32(0.0)
    cs = (_shift1(score, -1, z) + score) + _shift1(score, 1, z)
    sm = ((_shift0(cs, -1, z) + cs) + _shift0(cs, 1, z)) / 9.0

    lmax = _win17(_win17(sm, _shift0), _shift1)
    is_max = (sm == lmax) & inb

    smm = jnp.where(is_max, sm, 0.0)
    qmap = jnp.where(is_max, ((ri2 % 8) * 8 + ci2 % 8).astype(jnp.float32), 0.0)
    cntm = is_max.astype(jnp.float32)
    t64 = lax.broadcasted_iota(jnp.int32, (_HC, _H), 0)
    p512 = lax.broadcasted_iota(jnp.int32, (_HC, _H), 1)
    a = (p512 // 8 == t64).astype(jnp.float32)
    at = (r512 // 8 == c64).astype(jnp.float32)

    def tsum(x):
        return jax.lax.dot(jax.lax.dot(a, x, precision=_HI), at, precision=_HI)

    val_t = tsum(smm)
    q_t = tsum(qmap).astype(jnp.int32)
    cnt_t = tsum(cntm)
    present = cnt_t > 0.5
    ukey = jnp.where(
        present,
        lax.bitcast_convert_type(val_t, jnp.uint32) ^ jnp.uint32(0x80000000),
        jnp.uint32(0))
    tr = lax.broadcasted_iota(jnp.int32, (_HC, _HC), 0)
    tc = lax.broadcasted_iota(jnp.int32, (_HC, _HC), 1)
    gidx = (8 * tr + q_t // 8) * _W + 8 * tc + q_t % 8

    pvec = (lax.broadcasted_iota(jnp.int32, (8, 64), 0) * 64
            + lax.broadcasted_iota(jnp.int32, (8, 64), 1))
    upad = jnp.uint32(0x7FFFFFFF) - pvec.astype(jnp.uint32)

    def bit_body(k, t):
        t1 = t | (jnp.uint32(1) << jnp.uint32(31 - k))
        cnt = (jnp.sum((ukey >= t1).astype(jnp.int32))
               + jnp.sum((upad >= t1).astype(jnp.int32)))
        return jnp.where(cnt >= _K, t1, t)

    t_star = jax.lax.fori_loop(0, 32, bit_body, jnp.uint32(0))
    mask_c = ukey >= t_star
    mask_p = upad >= t_star

    lt64 = (lax.broadcasted_iota(jnp.int32, (_HC, _HC), 0)
            < lax.broadcasted_iota(jnp.int32, (_HC, _HC), 1)).astype(jnp.float32)
    ltr64 = (lax.broadcasted_iota(jnp.int32, (_HC, _HC), 1)
             < lax.broadcasted_iota(jnp.int32, (_HC, _HC), 0)).astype(jnp.float32)
    mc = mask_c.astype(jnp.float32)
    rowpfx = jax.lax.dot(mc, lt64, precision=_HI)
    rowsum = jnp.sum(mc, axis=1, keepdims=True)
    rowoff = jax.lax.dot(ltr64, rowsum, precision=_HI)
    pos_c = rowpfx + rowoff
    total_c = jnp.sum(mc)
    lt64p = lt64[:8, :]
    mp = mask_p.astype(jnp.float32)
    rowpfx_p = jax.lax.dot(mp, lt64, precision=_HI)
    rowsum_p = jnp.sum(mp, axis=1, keepdims=True)
    ltr8 = (lax.broadcasted_iota(jnp.int32, (8, 8), 1)
            < lax.broadcasted_iota(jnp.int32, (8, 8), 0)).astype(jnp.float32)
    rowoff_p = jax.lax.dot(ltr8, rowsum_p, precision=_HI)
    pos_p = rowpfx_p + rowoff_p + total_c

    khi_c = (ukey >> jnp.uint32(16)).astype(jnp.float32)
    klo_c = (ukey & jnp.uint32(0xFFFF)).astype(jnp.float32)
    gix_c = gidx.astype(jnp.float32)
    khi_p = (upad >> jnp.uint32(16)).astype(jnp.float32)
    klo_p = (upad & jnp.uint32(0xFFFF)).astype(jnp.float32)
    gix_p = pvec.astype(jnp.float32)

    pos_ci = pos_c.astype(jnp.int32)
    pos_pi = pos_p.astype(jnp.int32)
    chunks = []
    for k in range(4):
        s_c = (lax.broadcasted_iota(jnp.int32, (128, _HC, _HC), 0)
               + jnp.int32(128 * k))
        hit_c = ((pos_ci[None] == s_c) & mask_c[None]).astype(jnp.float32)
        s_p = (lax.broadcasted_iota(jnp.int32, (128, 8, 64), 0)
               + jnp.int32(128 * k))
        hit_p = ((pos_pi[None] == s_p) & mask_p[None]).astype(jnp.float32)

        def red(hc, hp, vc, vp):
            tc_ = jnp.sum(jnp.sum(hc * vc[None], axis=2), axis=1, keepdims=True)
            tp_ = jnp.sum(jnp.sum(hp * vp[None], axis=2), axis=1, keepdims=True)
            return tc_ + tp_

        chunks.append((red(hit_c, hit_p, khi_c, khi_p),
                       red(hit_c, hit_p, klo_c, klo_p),
                       red(hit_c, hit_p, gix_c, gix_p)))
    sel_khi = jnp.concatenate([c[0] for c in chunks], axis=0)
    sel_klo = jnp.concatenate([c[1] for c in chunks], axis=0)
    sel_gix = jnp.concatenate([c[2] for c in chunks], axis=0)
    sel_ukey = ((sel_khi.astype(jnp.uint32) << jnp.uint32(16))
                | sel_klo.astype(jnp.uint32))
    sel_gidx = sel_gix.astype(jnp.int32)

    rk = []
    for k in range(4):
        ks = sel_ukey[128 * k:128 * (k + 1)]
        ix = sel_gidx[128 * k:128 * (k + 1)]
        gt_c = (ukey[None] > ks[:, :, None]).astype(jnp.float32)
        tie_c = ((ukey[None] == ks[:, :, None])
                 & (gidx[None] < ix[:, :, None])).astype(jnp.float32)
        gt_p = (upad[None] > ks[:, :, None]).astype(jnp.float32)
        tie_p = ((upad[None] == ks[:, :, None])
                 & (pvec[None] < ix[:, :, None])).astype(jnp.float32)
        r_c = jnp.sum(jnp.sum(gt_c + tie_c, axis=2), axis=1, keepdims=True)
        r_p = jnp.sum(jnp.sum(gt_p + tie_p, axis=2), axis=1, keepdims=True)
        rk.append(r_c + r_p)
    rank = jnp.concatenate(rk, axis=0)

    t2d = lax.broadcasted_iota(jnp.int32, (_K, _K), 1)
    wm = (rank.astype(jnp.int32) == t2d).astype(jnp.float32)
    xv = (sel_gidx % _W).astype(jnp.float32)
    yv = (sel_gidx // _W).astype(jnp.float32)
    skey_i = lax.bitcast_convert_type(
        sel_ukey ^ jnp.uint32(0x80000000), jnp.int32)
    scv = jnp.where(skey_i >= 0,
                    lax.bitcast_convert_type(skey_i, jnp.float32), 0.0)

    def scat(v):
        return jax.lax.dot_general(wm, v, (((0,), (0,)), ((), ())),
                                   precision=_HI)

    x_ref[0] = scat(xv)
    y_ref[0] = scat(yv)
    s_ref[0] = scat(scv)


@jax.jit
def kernel(heatmap):
    b = heatmap.shape[0]
    out = pl.pallas_call(
        _kernel_body,
        grid=(b,),
        in_specs=[pl.BlockSpec((1, 65, 64, 64), lambda i: (i, 0, 0, 0))],
        out_specs=[
            pl.BlockSpec((1, _K, 1), lambda i: (i, 0, 0)),
            pl.BlockSpec((1, _K, 1), lambda i: (i, 0, 0)),
            pl.BlockSpec((1, _K, 1), lambda i: (i, 0, 0)),
        ],
        out_shape=[
            jax.ShapeDtypeStruct((b, _K, 1), jnp.float32),
            jax.ShapeDtypeStruct((b, _K, 1), jnp.float32),
            jax.ShapeDtypeStruct((b, _K, 1), jnp.float32),
        ],
    )(heatmap)
    x, y, sc = out
    kp = jnp.concatenate([x, y], axis=-1)
    return kp, sc[:, :, 0]


# speedup vs baseline: 2.9234x; 1.0012x over previous
import functools

import jax
import jax.numpy as jnp
from jax import lax
from jax.experimental import pallas as pl
from jax.experimental.pallas import tpu as pltpu

_K = 512
_H = 512
_W = 512
_HC = 64
_BORDER = 16
_HI = jax.lax.Precision.HIGHEST


def _shift0(x, d, fill):
    if d == 0:
        return x
    f = jnp.full((abs(d),) + x.shape[1:], fill, x.dtype)
    if d > 0:
        return jnp.concatenate([x[d:], f], axis=0)
    return jnp.concatenate([f, x[:d]], axis=0)


def _shift1(x, d, fill):
    if d == 0:
        return x
    f = jnp.full((x.shape[0], abs(d)) + x.shape[2:], fill, x.dtype)
    if d > 0:
        return jnp.concatenate([x[:, d:], f], axis=1)
    return jnp.concatenate([f, x[:, :d]], axis=1)


def _win17(x, shift):
    ninf = jnp.float32(-jnp.inf)
    w = x
    for d in range(1, 9):
        w = jnp.maximum(w, shift(x, d, ninf))
        w = jnp.maximum(w, shift(x, -d, ninf))
    return w


def _sum65(es):
    while len(es) > 1:
        es = [es[i] + es[i + 1] if i + 1 < len(es) else es[i]
              for i in range(0, len(es), 2)]
    return es[0]


def _kernel_body(h_ref, x_ref, y_ref, s_ref):
    h = h_ref[0]

    m = jnp.max(h, axis=0)
    e = jnp.exp(h - m[None])
    s = _sum65([e[c] for c in range(65)])
    prob = e[:64] / s[None]

    r512 = lax.broadcasted_iota(jnp.int32, (_H, _HC), 0)
    c64 = lax.broadcasted_iota(jnp.int32, (_H, _HC), 1)
    r64 = lax.broadcasted_iota(jnp.int32, (_HC, _W), 0)
    c512 = lax.broadcasted_iota(jnp.int32, (_HC, _W), 1)
    score = jnp.zeros((_H, _W), jnp.float32)
    for i in range(8):
        mi = jnp.zeros((_HC, _W), jnp.float32)
        for j in range(8):
            cjt = (c512 == 8 * r64 + j).astype(jnp.float32)
            mi = mi + jax.lax.dot(prob[8 * i + j], cjt, precision=_HI)
        ri = (r512 == 8 * c64 + i).astype(jnp.float32)
        score = score + jax.lax.dot(ri, mi, precision=_HI)

    ri2 = lax.broadcasted_iota(jnp.int32, (_H, _W), 0)
    ci2 = lax.broadcasted_iota(jnp.int32, (_H, _W), 1)
    inb = ((ri2 >= _BORDER) & (ri2 < _H - _BORDER)
           & (ci2 >= _BORDER) & (ci2 < _W - _BORDER))
    score = jnp.where(inb, score, 0.0)
    z = jnp.float32(0.0)
    cs = (_shift1(score, -1, z) + score) + _shift1(score, 1, z)
    sm = ((cs + _shift0(cs, 1, z)) + _shift0(cs, -1, z)) / 9.0

    lmax = _win17(_win17(sm, _shift0), _shift1)
    is_max = (sm == lmax) & inb

    smm = jnp.where(is_max, sm, 0.0)
    qmap = jnp.where(is_max, ((ri2 % 8) * 8 + ci2 % 8).astype(jnp.float32), 0.0)
    cntm = is_max.astype(jnp.float32)
    t64 = lax.broadcasted_iota(jnp.int32, (_HC, _H), 0)
    p512 = lax.broadcasted_iota(jnp.int32, (_HC, _H), 1)
    a = (p512 // 8 == t64).astype(jnp.float32)
    at = (r512 // 8 == c64).astype(jnp.float32)

    def tsum(x):
        return jax.lax.dot(jax.lax.dot(a, x, precision=_HI), at, precision=_HI)

    val_t = tsum(smm)
    q_t = tsum(qmap).astype(jnp.int32)
    cnt_t = tsum(cntm)
    present = cnt_t > 0.5
    ukey = jnp.where(
        present,
        lax.bitcast_convert_type(val_t, jnp.uint32) ^ jnp.uint32(0x80000000),
        jnp.uint32(0))
    tr = lax.broadcasted_iota(jnp.int32, (_HC, _HC), 0)
    tc = lax.broadcasted_iota(jnp.int32, (_HC, _HC), 1)
    gidx = (8 * tr + q_t // 8) * _W + 8 * tc + q_t % 8

    pvec = (lax.broadcasted_iota(jnp.int32, (8, 64), 0) * 64
            + lax.broadcasted_iota(jnp.int32, (8, 64), 1))
    upad = jnp.uint32(0x7FFFFFFF) - pvec.astype(jnp.uint32)

    def bit_body(k, t):
        t1 = t | (jnp.uint32(1) << jnp.uint32(31 - k))
        cnt = (jnp.sum((ukey >= t1).astype(jnp.int32))
               + jnp.sum((upad >= t1).astype(jnp.int32)))
        return jnp.where(cnt >= _K, t1, t)

    t_star = jax.lax.fori_loop(0, 32, bit_body, jnp.uint32(0))
    mask_c = ukey >= t_star
    mask_p = upad >= t_star

    lt64 = (lax.broadcasted_iota(jnp.int32, (_HC, _HC), 0)
            < lax.broadcasted_iota(jnp.int32, (_HC, _HC), 1)).astype(jnp.float32)
    ltr64 = (lax.broadcasted_iota(jnp.int32, (_HC, _HC), 1)
             < lax.broadcasted_iota(jnp.int32, (_HC, _HC), 0)).astype(jnp.float32)
    mc = mask_c.astype(jnp.float32)
    rowpfx = jax.lax.dot(mc, lt64, precision=_HI)
    rowsum = jnp.sum(mc, axis=1, keepdims=True)
    rowoff = jax.lax.dot(ltr64, rowsum, precision=_HI)
    pos_c = rowpfx + rowoff
    total_c = jnp.sum(mc)
    lt64p = lt64[:8, :]
    mp = mask_p.astype(jnp.float32)
    rowpfx_p = jax.lax.dot(mp, lt64, precision=_HI)
    rowsum_p = jnp.sum(mp, axis=1, keepdims=True)
    ltr8 = (lax.broadcasted_iota(jnp.int32, (8, 8), 1)
            < lax.broadcasted_iota(jnp.int32, (8, 8), 0)).astype(jnp.float32)
    rowoff_p = jax.lax.dot(ltr8, rowsum_p, precision=_HI)
    pos_p = rowpfx_p + rowoff_p + total_c

    khi_c = (ukey >> jnp.uint32(16)).astype(jnp.float32)
    klo_c = (ukey & jnp.uint32(0xFFFF)).astype(jnp.float32)
    gix_c = gidx.astype(jnp.float32)
    khi_p = (upad >> jnp.uint32(16)).astype(jnp.float32)
    klo_p = (upad & jnp.uint32(0xFFFF)).astype(jnp.float32)
    gix_p = pvec.astype(jnp.float32)

    pos_ci = pos_c.astype(jnp.int32)
    pos_pi = pos_p.astype(jnp.int32)
    chunks = []
    for k in range(4):
        s_c = (lax.broadcasted_iota(jnp.int32, (128, _HC, _HC), 0)
               + jnp.int32(128 * k))
        hit_c = ((pos_ci[None] == s_c) & mask_c[None]).astype(jnp.float32)
        s_p = (lax.broadcasted_iota(jnp.int32, (128, 8, 64), 0)
               + jnp.int32(128 * k))
        hit_p = ((pos_pi[None] == s_p) & mask_p[None]).astype(jnp.float32)

        def red(hc, hp, vc, vp):
            tc_ = jnp.sum(jnp.sum(hc * vc[None], axis=2), axis=1, keepdims=True)
            tp_ = jnp.sum(jnp.sum(hp * vp[None], axis=2), axis=1, keepdims=True)
            return tc_ + tp_

        chunks.append((red(hit_c, hit_p, khi_c, khi_p),
                       red(hit_c, hit_p, klo_c, klo_p),
                       red(hit_c, hit_p, gix_c, gix_p)))
    sel_khi = jnp.concatenate([c[0] for c in chunks], axis=0)
    sel_klo = jnp.concatenate([c[1] for c in chunks], axis=0)
    sel_gix = jnp.concatenate([c[2] for c in chunks], axis=0)
    sel_ukey = ((sel_khi.astype(jnp.uint32) << jnp.uint32(16))
                | sel_klo.astype(jnp.uint32))
    sel_gidx = sel_gix.astype(jnp.int32)

    rk = []
    for k in range(4):
        ks = sel_ukey[128 * k:128 * (k + 1)]
        ix = sel_gidx[128 * k:128 * (k + 1)]
        gt_c = (ukey[None] > ks[:, :, None]).astype(jnp.float32)
        tie_c = ((ukey[None] == ks[:, :, None])
                 & (gidx[None] < ix[:, :, None])).astype(jnp.float32)
        gt_p = (upad[None] > ks[:, :, None]).astype(jnp.float32)
        tie_p = ((upad[None] == ks[:, :, None])
                 & (pvec[None] < ix[:, :, None])).astype(jnp.float32)
        r_c = jnp.sum(jnp.sum(gt_c + tie_c, axis=2), axis=1, keepdims=True)
        r_p = jnp.sum(jnp.sum(gt_p + tie_p, axis=2), axis=1, keepdims=True)
        rk.append(r_c + r_p)
    rank = jnp.concatenate(rk, axis=0)

    t2d = lax.broadcasted_iota(jnp.int32, (_K, _K), 1)
    wm = (rank.astype(jnp.int32) == t2d).astype(jnp.float32)
    xv = (sel_gidx % _W).astype(jnp.float32)
    yv = (sel_gidx // _W).astype(jnp.float32)
    skey_i = lax.bitcast_convert_type(
        sel_ukey ^ jnp.uint32(0x80000000), jnp.int32)
    scv = jnp.where(skey_i >= 0,
                    lax.bitcast_convert_type(skey_i, jnp.float32), 0.0)

    def scat(v):
        return jax.lax.dot_general(wm, v, (((0,), (0,)), ((), ())),
                                   precision=_HI)

    x_ref[0] = scat(xv)
    y_ref[0] = scat(yv)
    s_ref[0] = scat(scv)


@jax.jit
def kernel(heatmap):
    b = heatmap.shape[0]
    out = pl.pallas_call(
        _kernel_body,
        grid=(b,),
        in_specs=[pl.BlockSpec((1, 65, 64, 64), lambda i: (i, 0, 0, 0))],
        out_specs=[
            pl.BlockSpec((1, _K, 1), lambda i: (i, 0, 0)),
            pl.BlockSpec((1, _K, 1), lambda i: (i, 0, 0)),
            pl.BlockSpec((1, _K, 1), lambda i: (i, 0, 0)),
        ],
        out_shape=[
            jax.ShapeDtypeStruct((b, _K, 1), jnp.float32),
            jax.ShapeDtypeStruct((b, _K, 1), jnp.float32),
            jax.ShapeDtypeStruct((b, _K, 1), jnp.float32),
        ],
    )(heatmap)
    x, y, sc = out
    kp = jnp.concatenate([x, y], axis=-1)
    return kp, sc[:, :, 0]
